# Initial kernel scaffold; baseline (speedup 1.0000x reference)
import jax
import jax.numpy as jnp
from jax.experimental import pallas as pl


def kernel(pos_u, pos_v, neg_v, input_emb, output_emb):
    raise NotImplementedError("write your pallas kernel here")


# speedup vs baseline: 1.6882x; 1.6882x over previous
import functools

import jax
import jax.numpy as jnp
from jax import lax
from jax.experimental import pallas as pl
from jax.experimental.pallas import tpu as pltpu
from jax.experimental.pallas import tpu_sc as plsc

B = 16384
D = 64
NEG = 5
L = 16
NC = 2
NS = 16
NW = NC * NS
ITEMS = B // NW
BLK = 128
NBLK = ITEMS // BLK

_mesh = plsc.VectorSubcoreMesh(core_axis_name="c", subcore_axis_name="s")


@functools.partial(
    pl.kernel,
    mesh=_mesh,
    compiler_params=pltpu.CompilerParams(
        needs_layout_passes=False, use_tc_tiling_on_sc=False),
    out_type=jax.ShapeDtypeStruct(((1 + NEG) * B,), jnp.float32),
    scratch_types=[
        pltpu.VMEM((BLK,), jnp.int32),
        pltpu.VMEM((BLK,), jnp.int32),
        pltpu.VMEM((NEG * BLK,), jnp.int32),
        pltpu.VMEM((BLK, D), jnp.float32),
        pltpu.VMEM((BLK, D), jnp.float32),
        pltpu.VMEM((NEG * BLK, D), jnp.float32),
        pltpu.VMEM(((1 + NEG) * BLK,), jnp.float32),
        pltpu.VMEM((1 + NEG, L, L), jnp.float32),
        pltpu.SemaphoreType.DMA,
    ],
)
def _scores_kernel(pos_u, pos_v, neg_vt, in_emb, out_emb, out,
                   uidx, vidx, nidx, urows, vrows, nrows, scores, ptile, sem):
    wid = lax.axis_index("s") * NC + lax.axis_index("c")

    for blk in range(NBLK):
        base = wid * ITEMS + blk * BLK
        pltpu.sync_copy(pos_u.at[pl.ds(base, BLK)], uidx)
        pltpu.sync_copy(pos_v.at[pl.ds(base, BLK)], vidx)
        for k in range(NEG):
            pltpu.sync_copy(neg_vt.at[pl.ds(k * B + base, BLK)],
                            nidx.at[pl.ds(k * BLK, BLK)])
        cps = [
            pltpu.async_copy(in_emb.at[uidx], urows, sem),
            pltpu.async_copy(out_emb.at[vidx], vrows, sem),
        ]
        for k in range(NEG):
            cps.append(pltpu.async_copy(
                out_emb.at[nidx.at[pl.ds(k * BLK, BLK)]],
                nrows.at[pl.ds(k * BLK, BLK)], sem))
        for cp in cps:
            cp.wait()

        lanes = lax.iota(jnp.int32, L)

        def group(g, _):
            for ii in range(L):
                i = g * L + ii
                u = [urows[i, pl.ds(16 * j, 16)] for j in range(D // L)]
                v = [vrows[i, pl.ds(16 * j, 16)] for j in range(D // L)]
                p = u[0] * v[0]
                for j in range(1, D // L):
                    p = p + u[j] * v[j]
                ptile[0, ii] = p
                for k in range(NEG):
                    w = [nrows[k * BLK + i, pl.ds(16 * j, 16)]
                         for j in range(D // L)]
                    q = u[0] * w[0]
                    for j in range(1, D // L):
                        q = q + u[j] * w[j]
                    ptile[1 + k, ii] = q
            for j in range(1 + NEG):
                jfull = jnp.full((L,), j, jnp.int32)
                acc = plsc.load_gather(ptile, [jfull, lanes, jnp.zeros((L,), jnp.int32)])
                for c in range(1, L):
                    acc = acc + plsc.load_gather(
                        ptile, [jfull, lanes, jnp.full((L,), c, jnp.int32)])
                scores[pl.ds(j * BLK + g * L, L)] = acc
            return 0

        lax.fori_loop(0, BLK // L, group, 0)

        for j in range(1 + NEG):
            pltpu.sync_copy(scores.at[pl.ds(j * BLK, BLK)],
                            out.at[pl.ds(j * B + base, BLK)])


def _loss_body(s_ref, o_ref):
    s = s_ref[...]
    row = lax.broadcasted_iota(jnp.int32, s.shape, 0)
    x = jnp.where(row == 0, s, -s)
    l = jnp.minimum(x, 0.0) - jnp.log1p(jnp.exp(-jnp.abs(x)))
    o_ref[0, 0] = -jnp.sum(l) / B


_loss_call = pl.pallas_call(
    _loss_body,
    out_shape=jax.ShapeDtypeStruct((1, 1), jnp.float32),
    out_specs=pl.BlockSpec(memory_space=pltpu.SMEM),
)


def kernel(pos_u, pos_v, neg_v, input_emb, output_emb):
    neg_flat = neg_v.T.reshape(-1)
    scores = _scores_kernel(pos_u, pos_v, neg_flat, input_emb, output_emb)
    return _loss_call(scores.reshape(1 + NEG, B))[0, 0]
